# baseline (speedup 1.0000x reference)
import functools

import jax
import jax.numpy as jnp
from jax import lax
from jax.experimental import pallas as pl
from jax.experimental.pallas import tpu as pltpu
from jax.experimental.pallas import tpu_sc as plsc

_LANES = 128


@functools.cache
def _build(batch, vocab, embed):
    info = plsc.get_sparse_core_info()
    nc, ns = info.num_cores, info.num_subcores
    nw = nc * ns
    b_per_w = batch // nw
    assert batch % (8 * nw) == 0

    mesh = plsc.VectorSubcoreMesh(core_axis_name="c", subcore_axis_name="s")

    @functools.partial(
        pl.kernel,
        mesh=mesh,
        out_type=jax.ShapeDtypeStruct((batch, _LANES), jnp.float32),
        scratch_types=[
            pltpu.VMEM((b_per_w,), jnp.int32),
            pltpu.VMEM((b_per_w, _LANES), jnp.float32),
            pltpu.SemaphoreType.DMA,
        ],
    )
    def gather_kernel(idx_hbm, table2_hbm, out_hbm, idx_v, rows_v, gsem):
        wid = lax.axis_index("s") * nc + lax.axis_index("c")
        base = wid * b_per_w
        pltpu.sync_copy(idx_hbm.at[pl.ds(base, b_per_w)], idx_v)

        def body(c, _):
            vec = idx_v[pl.ds(c * 16, 16)]
            for j in range(16):
                r = vec[j]
                pltpu.async_copy(
                    table2_hbm.at[r // 2, pl.ds((r % 2) * embed, embed)],
                    rows_v.at[c * 16 + j, pl.ds(0, embed)],
                    gsem,
                )
            return 0

        lax.fori_loop(0, b_per_w // 16, body, 0)

        def drain(c, _):
            vec = idx_v[pl.ds(c * 16, 16)]
            for j in range(16):
                r = vec[j]
                pltpu.make_async_copy(
                    table2_hbm.at[r // 2, pl.ds((r % 2) * embed, embed)],
                    rows_v.at[c * 16 + j, pl.ds(0, embed)],
                    gsem,
                ).wait()
            return 0

        lax.fori_loop(0, b_per_w // 16, drain, 0)
        pltpu.sync_copy(rows_v, out_hbm.at[pl.ds(base, b_per_w)])

    return gather_kernel


def kernel(inputs, embeddings):
    vocab, embed = embeddings.shape
    (batch,) = inputs.shape
    table2 = embeddings.reshape(vocab // 2, 2 * embed)
    wide = _build(batch, vocab, embed)(inputs, table2)
    return wide[:, :embed]


# speedup vs baseline: 1.7261x; 1.7261x over previous
import functools

import jax
import jax.numpy as jnp
from jax import lax
from jax.experimental import pallas as pl
from jax.experimental.pallas import tpu as pltpu
from jax.experimental.pallas import tpu_sc as plsc

_LANES = 128


@functools.cache
def _build(batch, vocab, embed):
    info = plsc.get_sparse_core_info()
    nc, ns = info.num_cores, info.num_subcores
    nw = nc * ns
    b_per_w = batch // nw
    assert batch % (8 * nw) == 0

    mesh = plsc.VectorSubcoreMesh(core_axis_name="c", subcore_axis_name="s")

    @functools.partial(
        pl.kernel,
        mesh=mesh,
        out_type=jax.ShapeDtypeStruct((batch, _LANES), jnp.float32),
        scratch_types=[
            pltpu.VMEM((b_per_w,), jnp.int32),
            pltpu.VMEM((b_per_w, _LANES), jnp.float32),
            pltpu.SemaphoreType.DMA,
        ],
    )
    def gather_kernel(idx_hbm, table_hbm, out_hbm, idx_v, rows_v, gsem):
        wid = lax.axis_index("s") * nc + lax.axis_index("c")
        base = wid * b_per_w
        pltpu.sync_copy(idx_hbm.at[pl.ds(base, b_per_w)], idx_v)

        def body(c, _):
            vec = idx_v[pl.ds(c * 16, 16)]
            for j in range(16):
                r = vec[j]
                pltpu.async_copy(
                    table_hbm.at[r],
                    rows_v.at[c * 16 + j, pl.ds(0, embed)],
                    gsem,
                )
            return 0

        lax.fori_loop(0, b_per_w // 16, body, 0)

        pltpu.make_async_copy(
            out_hbm.at[pl.ds(0, b_per_w // 2)],
            rows_v.at[pl.ds(0, b_per_w // 2), :],
            gsem,
        ).wait()
        pltpu.sync_copy(rows_v, out_hbm.at[pl.ds(base, b_per_w)])

    return gather_kernel


def kernel(inputs, embeddings):
    vocab, embed = embeddings.shape
    (batch,) = inputs.shape
    wide = _build(batch, vocab, embed)(inputs, embeddings)
    return wide[:, :embed]
